# baseline (speedup 1.0000x reference)
import jax
import jax.numpy as jnp
from jax import lax
from jax.experimental import pallas as pl
from jax.experimental.pallas import tpu as pltpu

B = 16
N = 16
D = 128
SEQ = 2048
KW = 2 * N * N
SEQ2 = SEQ // 2
KWL = KW // 2
NCHUNK = 4
CHUNK = SEQ2 // NCHUNK
KWR = SEQ // (N * NCHUNK)


def _update_kernel(pos_ref, k_val, v_val, kw_val, k_in, v_in, kw_in,
                   k_out, v_out, kw_out):
    i = pl.program_id(0)
    j = pl.program_id(1)
    pos = pos_ref[0]
    p2 = pos // 2
    half = pos % 2

    k_out[...] = k_in[...]
    v_out[...] = v_in[...]
    kw_out[...] = kw_in[...]

    @pl.when(j == p2 // CHUNK)
    def _():
        row = lax.broadcasted_iota(jnp.int32, (1, CHUNK, D), 1) + j * CHUNK
        lane = lax.broadcasted_iota(jnp.int32, (1, CHUNK, D), 2)
        mask = (row == p2) & (lane // 64 == half)
        k_out[...] = jnp.where(mask, k_val[...], k_in[...])
        v_out[...] = jnp.where(mask, v_val[...], v_in[...])

    cell = (i % N) * NCHUNK + j

    @pl.when(cell == pos // KWR)
    def _():
        row = lax.broadcasted_iota(jnp.int32, (1, KWR, KWL), 1) + cell * KWR
        kw_out[...] = jnp.where(row == pos, kw_val[...], kw_in[...])


_GRID_SPEC = pltpu.PrefetchScalarGridSpec(
    num_scalar_prefetch=1,
    grid=(B * N, NCHUNK),
    in_specs=[
        pl.BlockSpec((1, 1, D), lambda i, j, pos: (i, 0, 0)),
        pl.BlockSpec((1, 1, D), lambda i, j, pos: (i, 0, 0)),
        pl.BlockSpec((1, 1, KWL), lambda i, j, pos: (i // N, 0, 0)),
        pl.BlockSpec((1, CHUNK, D), lambda i, j, pos: (i, j, 0)),
        pl.BlockSpec((1, CHUNK, D), lambda i, j, pos: (i, j, 0)),
        pl.BlockSpec((1, KWR, KWL),
                     lambda i, j, pos: (i // N, (i % N) * NCHUNK + j, 0)),
    ],
    out_specs=[
        pl.BlockSpec((1, CHUNK, D), lambda i, j, pos: (i, j, 0)),
        pl.BlockSpec((1, CHUNK, D), lambda i, j, pos: (i, j, 0)),
        pl.BlockSpec((1, KWR, KWL),
                     lambda i, j, pos: (i // N, (i % N) * NCHUNK + j, 0)),
    ],
)


def _as_i32(x, shape):
    return lax.bitcast_convert_type(x.reshape(*shape, 2), jnp.int32)


def kernel(input_pos, k_val, v_val, kw_val, k_cache, v_cache, kw_cache):
    pos = input_pos.astype(jnp.int32) % SEQ
    k32 = _as_i32(k_cache, (B * N, SEQ2, D))
    v32 = _as_i32(v_cache, (B * N, SEQ2, D))
    kw32 = _as_i32(kw_cache, (B, SEQ, KWL))
    kv = _as_i32(k_val, (B * N, 1, D // 2))
    vv = _as_i32(v_val, (B * N, 1, D // 2))
    kwv = _as_i32(kw_val, (B, 1, KWL))
    kv = jnp.concatenate([kv, kv], axis=-1)
    vv = jnp.concatenate([vv, vv], axis=-1)

    k_out, v_out, kw_out = pl.pallas_call(
        _update_kernel,
        grid_spec=_GRID_SPEC,
        out_shape=[
            jax.ShapeDtypeStruct((B * N, SEQ2, D), jnp.int32),
            jax.ShapeDtypeStruct((B * N, SEQ2, D), jnp.int32),
            jax.ShapeDtypeStruct((B, SEQ, KWL), jnp.int32),
        ],
    )(pos, kv, vv, kwv, k32, v32, kw32)

    dt = k_cache.dtype
    return (
        lax.bitcast_convert_type(k_out, dt).reshape(B, N, SEQ, D),
        lax.bitcast_convert_type(v_out, dt).reshape(B, N, SEQ, D),
        lax.bitcast_convert_type(kw_out, dt).reshape(B, SEQ, 2, N, N),
    )


# speedup vs baseline: 1.2106x; 1.2106x over previous
import jax
import jax.numpy as jnp
from jax import lax
from jax.experimental import pallas as pl
from jax.experimental.pallas import tpu as pltpu

B = 16
N = 16
D = 128
SEQ = 2048
KW = 2 * N * N
BN = B * N
GRP = 16
NG = SEQ // GRP
KV_CH = 8
KW_CH = 2
KV_ROWS = BN // KV_CH
KW_ROWS = B // KW_CH


def _update_kernel(pos_ref, k_grp, v_grp, kw_grp, k_in, v_in, kw_in,
                   k_out, v_out, kw_out, bulk_sem, row_sem):
    g = pos_ref[0] // GRP
    bulk = []
    for c in range(KV_CH):
        s = pl.ds(c * KV_ROWS, KV_ROWS)
        bulk.append(pltpu.make_async_copy(
            k_in.at[s], k_out.at[s], bulk_sem.at[c]))
        bulk.append(pltpu.make_async_copy(
            v_in.at[s], v_out.at[s], bulk_sem.at[KV_CH + c]))
    for c in range(KW_CH):
        s = pl.ds(c * KW_ROWS, KW_ROWS)
        bulk.append(pltpu.make_async_copy(
            kw_in.at[s], kw_out.at[s], bulk_sem.at[2 * KV_CH + c]))
    for cp in bulk:
        cp.start()
    for cp in bulk:
        cp.wait()

    gs = pl.ds(g, 1)
    rows = [
        pltpu.make_async_copy(k_grp, k_out.at[:, gs], row_sem.at[0]),
        pltpu.make_async_copy(v_grp, v_out.at[:, gs], row_sem.at[1]),
        pltpu.make_async_copy(kw_grp, kw_out.at[:, gs], row_sem.at[2]),
    ]
    for cp in rows:
        cp.start()
    for cp in rows:
        cp.wait()


_GRID_SPEC = pltpu.PrefetchScalarGridSpec(
    num_scalar_prefetch=1,
    grid=(1,),
    in_specs=[pl.BlockSpec(memory_space=pl.ANY)] * 6,
    out_specs=[pl.BlockSpec(memory_space=pl.ANY)] * 3,
    scratch_shapes=[
        pltpu.SemaphoreType.DMA((2 * KV_CH + KW_CH,)),
        pltpu.SemaphoreType.DMA((3,)),
    ],
)


def kernel(input_pos, k_val, v_val, kw_val, k_cache, v_cache, kw_cache):
    pos = input_pos.astype(jnp.int32) % SEQ
    base = (pos[0] // GRP) * GRP
    dt = k_cache.dtype

    row_ids = base + lax.broadcasted_iota(jnp.int32, (1, GRP, 1), 1)
    k3 = k_cache.reshape(BN, SEQ, D)
    v3 = v_cache.reshape(BN, SEQ, D)
    kw3 = kw_cache.reshape(B, SEQ, KW)
    k_grp = jnp.where(row_ids == pos[0],
                      k_val.reshape(BN, 1, D),
                      lax.dynamic_slice(k3, (0, base, 0), (BN, GRP, D)))
    v_grp = jnp.where(row_ids == pos[0],
                      v_val.reshape(BN, 1, D),
                      lax.dynamic_slice(v3, (0, base, 0), (BN, GRP, D)))
    kw_grp = jnp.where(row_ids == pos[0],
                       kw_val.reshape(B, 1, KW),
                       lax.dynamic_slice(kw3, (0, base, 0), (B, GRP, KW)))

    bc = lambda x: lax.bitcast_convert_type(x, jnp.bfloat16)
    k_out, v_out, kw_out = pl.pallas_call(
        _update_kernel,
        grid_spec=_GRID_SPEC,
        out_shape=[
            jax.ShapeDtypeStruct((BN, NG, GRP, D), jnp.bfloat16),
            jax.ShapeDtypeStruct((BN, NG, GRP, D), jnp.bfloat16),
            jax.ShapeDtypeStruct((B, NG, GRP, KW), jnp.bfloat16),
        ],
    )(
        pos,
        bc(k_grp.reshape(BN, 1, GRP, D)),
        bc(v_grp.reshape(BN, 1, GRP, D)),
        bc(kw_grp.reshape(B, 1, GRP, KW)),
        bc(k3.reshape(BN, NG, GRP, D)),
        bc(v3.reshape(BN, NG, GRP, D)),
        bc(kw3.reshape(B, NG, GRP, KW)),
    )
    return (
        lax.bitcast_convert_type(k_out, dt).reshape(B, N, SEQ, D),
        lax.bitcast_convert_type(v_out, dt).reshape(B, N, SEQ, D),
        lax.bitcast_convert_type(kw_out, dt).reshape(B, SEQ, 2, N, N),
    )


# speedup vs baseline: 12.8525x; 10.6163x over previous
import jax
import jax.numpy as jnp
from jax import lax
from jax.experimental import pallas as pl
from jax.experimental.pallas import tpu as pltpu

B = 16
N = 16
D = 128
SEQ = 2048
KW = 2 * N * N
BN = B * N
GRP = 16
NG = SEQ // GRP
G = 32
KV_ROWS = BN * SEQ // G
KW_ROWS = B * SEQ // G


def _copy_kernel(k_in, v_in, kw_in, k_out, v_out, kw_out):
    k_out[...] = k_in[...]
    v_out[...] = v_in[...]
    kw_out[...] = kw_in[...]


_COPY_SPEC = dict(
    grid=(G,),
    in_specs=[
        pl.BlockSpec((KV_ROWS, D), lambda i: (i, 0)),
        pl.BlockSpec((KV_ROWS, D), lambda i: (i, 0)),
        pl.BlockSpec((KW_ROWS, KW), lambda i: (i, 0)),
    ],
    out_specs=[
        pl.BlockSpec((KV_ROWS, D), lambda i: (i, 0)),
        pl.BlockSpec((KV_ROWS, D), lambda i: (i, 0)),
        pl.BlockSpec((KW_ROWS, KW), lambda i: (i, 0)),
    ],
)


def _blend_kernel(pos_ref, k_val, v_val, kw_val, k_in, v_in, kw_in,
                  k_out, v_out, kw_out):
    sub = lax.broadcasted_iota(jnp.int32, (1, 1, GRP, 1), 2)
    hit = sub == pos_ref[0] % GRP
    k_out[...] = jnp.where(hit, k_val[...], k_in[...])
    v_out[...] = jnp.where(hit, v_val[...], v_in[...])
    kw_out[...] = jnp.where(hit, kw_val[...], kw_in[...])


_BLEND_SPEC = pltpu.PrefetchScalarGridSpec(
    num_scalar_prefetch=1,
    grid=(1,),
    in_specs=[
        pl.BlockSpec((BN, 1, 1, D), lambda i, pos: (0, 0, 0, 0)),
        pl.BlockSpec((BN, 1, 1, D), lambda i, pos: (0, 0, 0, 0)),
        pl.BlockSpec((B, 1, 1, KW), lambda i, pos: (0, 0, 0, 0)),
        pl.BlockSpec((BN, 1, GRP, D), lambda i, pos: (0, pos[0] // GRP, 0, 0)),
        pl.BlockSpec((BN, 1, GRP, D), lambda i, pos: (0, pos[0] // GRP, 0, 0)),
        pl.BlockSpec((B, 1, GRP, KW), lambda i, pos: (0, pos[0] // GRP, 0, 0)),
    ],
    out_specs=[
        pl.BlockSpec((BN, 1, GRP, D), lambda i, pos: (0, pos[0] // GRP, 0, 0)),
        pl.BlockSpec((BN, 1, GRP, D), lambda i, pos: (0, pos[0] // GRP, 0, 0)),
        pl.BlockSpec((B, 1, GRP, KW), lambda i, pos: (0, pos[0] // GRP, 0, 0)),
    ],
)


def kernel(input_pos, k_val, v_val, kw_val, k_cache, v_cache, kw_cache):
    pos = input_pos.astype(jnp.int32) % SEQ
    dt = k_cache.dtype
    bc = lambda x: lax.bitcast_convert_type(x, jnp.bfloat16)

    k_c, v_c, kw_c = pl.pallas_call(
        _copy_kernel,
        out_shape=[
            jax.ShapeDtypeStruct((BN * SEQ, D), jnp.bfloat16),
            jax.ShapeDtypeStruct((BN * SEQ, D), jnp.bfloat16),
            jax.ShapeDtypeStruct((B * SEQ, KW), jnp.bfloat16),
        ],
        **_COPY_SPEC,
    )(
        bc(k_cache.reshape(BN * SEQ, D)),
        bc(v_cache.reshape(BN * SEQ, D)),
        bc(kw_cache.reshape(B * SEQ, KW)),
    )
    k_out, v_out, kw_out = pl.pallas_call(
        _blend_kernel,
        grid_spec=_BLEND_SPEC,
        out_shape=[
            jax.ShapeDtypeStruct((BN, NG, GRP, D), jnp.bfloat16),
            jax.ShapeDtypeStruct((BN, NG, GRP, D), jnp.bfloat16),
            jax.ShapeDtypeStruct((B, NG, GRP, KW), jnp.bfloat16),
        ],
        input_output_aliases={4: 0, 5: 1, 6: 2},
    )(
        pos,
        bc(k_val.reshape(BN, 1, 1, D)),
        bc(v_val.reshape(BN, 1, 1, D)),
        bc(kw_val.reshape(B, 1, 1, KW)),
        k_c.reshape(BN, NG, GRP, D),
        v_c.reshape(BN, NG, GRP, D),
        kw_c.reshape(B, NG, GRP, KW),
    )
    return (
        lax.bitcast_convert_type(k_out, dt).reshape(B, N, SEQ, D),
        lax.bitcast_convert_type(v_out, dt).reshape(B, N, SEQ, D),
        lax.bitcast_convert_type(kw_out, dt).reshape(B, SEQ, 2, N, N),
    )
